# baseline (speedup 1.0000x reference)
import functools

import jax
import jax.numpy as jnp
from jax import lax
from jax.experimental import pallas as pl
from jax.experimental.pallas import tpu as pltpu
from jax.experimental.pallas import tpu_sc as plsc

BATCH = 16384
EMBED_DIM = 32

_NC = 2
_NS = 16
_NW = _NC * _NS
_BPW = BATCH // _NW
_CHUNK = 128
_NCHUNK = _BPW // _CHUNK


def _gather_body(uidx_hbm, iidx_hbm, utab, itab, uout, iout,
                 uidx_v, iidx_v, urows, irows, sem):
    c = lax.axis_index("c")
    s = lax.axis_index("s")
    wid = s * _NC + c
    base = wid * _BPW
    pltpu.sync_copy(uidx_hbm.at[wid], uidx_v)
    pltpu.sync_copy(iidx_hbm.at[wid], iidx_v)
    copies = []
    for j in range(_NCHUNK):
        copies.append(pltpu.async_copy(
            utab.at[uidx_v.at[j]], urows.at[pl.ds(j * _CHUNK, _CHUNK)], sem))
        copies.append(pltpu.async_copy(
            itab.at[iidx_v.at[j]], irows.at[pl.ds(j * _CHUNK, _CHUNK)], sem))
    for cp in copies:
        cp.wait()
    pltpu.sync_copy(urows, uout.at[pl.ds(base, _BPW)])
    pltpu.sync_copy(irows, iout.at[pl.ds(base, _BPW)])


@jax.jit
def _sc_gather(uidx3, iidx3, utab, itab):
    mesh = plsc.VectorSubcoreMesh(core_axis_name="c", subcore_axis_name="s")
    f = functools.partial(
        pl.kernel,
        mesh=mesh,
        out_type=[
            jax.ShapeDtypeStruct((BATCH, EMBED_DIM), jnp.float32),
            jax.ShapeDtypeStruct((BATCH, EMBED_DIM), jnp.float32),
        ],
        scratch_types=[
            pltpu.VMEM((_NCHUNK, _CHUNK), jnp.int32),
            pltpu.VMEM((_NCHUNK, _CHUNK), jnp.int32),
            pltpu.VMEM((_BPW, EMBED_DIM), jnp.float32),
            pltpu.VMEM((_BPW, EMBED_DIM), jnp.float32),
            pltpu.SemaphoreType.DMA,
        ],
        compiler_params=pltpu.CompilerParams(use_tc_tiling_on_sc=False),
    )(_gather_body)
    return f(uidx3, iidx3, utab, itab)


_BB = 1024
_NB = BATCH // _BB


def _mlp_body(u_ref, i_ref, w1u_ref, w1i_ref, b1_ref, w2_ref, b2_ref,
              w3_ref, b3_ref, out_ref):
    h1 = jnp.dot(u_ref[...], w1u_ref[...], preferred_element_type=jnp.float32)
    h1 = h1 + jnp.dot(i_ref[...], w1i_ref[...],
                      preferred_element_type=jnp.float32)
    h1 = jnp.maximum(h1 + b1_ref[...], 0.0)
    h2 = jnp.dot(h1, w2_ref[...], preferred_element_type=jnp.float32)
    h2 = jnp.maximum(h2 + b2_ref[...], 0.0)
    o = jnp.sum(h2 * w3_ref[...], axis=1) + b3_ref[0, 0]
    out_ref[:, 0] = o


@jax.jit
def _tc_mlp(u, i, w1u, w1i, b1, w2, b2, w3, b3):
    out2d = pl.pallas_call(
        _mlp_body,
        grid=(_NB,),
        in_specs=[
            pl.BlockSpec((_BB, EMBED_DIM), lambda b: (b, 0)),
            pl.BlockSpec((_BB, EMBED_DIM), lambda b: (b, 0)),
            pl.BlockSpec((EMBED_DIM, 64), lambda b: (0, 0)),
            pl.BlockSpec((EMBED_DIM, 64), lambda b: (0, 0)),
            pl.BlockSpec((1, 64), lambda b: (0, 0)),
            pl.BlockSpec((64, 32), lambda b: (0, 0)),
            pl.BlockSpec((1, 32), lambda b: (0, 0)),
            pl.BlockSpec((1, 32), lambda b: (0, 0)),
            pl.BlockSpec((1, 1), lambda b: (0, 0)),
        ],
        out_specs=pl.BlockSpec((_BB, 1), lambda b: (b, 0)),
        out_shape=jax.ShapeDtypeStruct((BATCH, 1), jnp.float32),
    )(u, i, w1u, w1i, b1, w2, b2, w3, b3)
    return out2d.reshape(BATCH)


def kernel(user, item, user_table, item_table, W1, b1, W2, b2, W3, b3):
    uidx3 = user.astype(jnp.int32).reshape(_NW, _NCHUNK, _CHUNK)
    iidx3 = item.astype(jnp.int32).reshape(_NW, _NCHUNK, _CHUNK)
    u_rows, i_rows = _sc_gather(uidx3, iidx3, user_table, item_table)
    w1u = W1[:, :EMBED_DIM].T
    w1i = W1[:, EMBED_DIM:].T
    return _tc_mlp(u_rows, i_rows, w1u, w1i, b1.reshape(1, 64),
                   W2.T, b2.reshape(1, 32), W3, b3.reshape(1, 1))
